# baseline (speedup 1.0000x reference)
import functools
import jax
import jax.numpy as jnp
import numpy as np
from jax import lax
from jax.experimental import pallas as pl
from jax.experimental.pallas import tpu as pltpu, tpu_sc as plsc

_SCALE = float(2.0 ** -2.5)

_B, _T = 128, 32768
_F = _T // 32
_RPW = 4
_mesh = plsc.VectorSubcoreMesh(core_axis_name="c", subcore_axis_name="s")


@functools.partial(
    pl.kernel,
    out_type=jax.ShapeDtypeStruct((_B, _F), jnp.float32),
    mesh=_mesh,
    scratch_types=[
        pltpu.VMEM((_T,), jnp.float32),
        pltpu.VMEM((_F,), jnp.float32),
    ],
    compiler_params=pltpu.CompilerParams(needs_layout_passes=False),
)
def _sc_reduce(x_hbm, o_hbm, buf, outb):
    wid = lax.axis_index("s") * 2 + lax.axis_index("c")
    r0 = wid * _RPW
    iota16 = lax.iota(jnp.int32, 16)
    for r in range(_RPW):
        row = r0 + r
        pltpu.sync_copy(x_hbm.at[row], buf)

        def g_body(g, _):
            col0 = g * 512 + iota16 * 32
            acc = jnp.zeros((16,), jnp.float32)
            for j in range(32):
                acc = acc + plsc.load_gather(buf, [col0 + j])
            outb[pl.ds(g * 16, 16)] = acc * _SCALE
            return 0

        lax.fori_loop(0, _F // 16, g_body, 0)
        pltpu.sync_copy(outb, o_hbm.at[row])


def kernel(x):
    return _sc_reduce(x)[:, :, None]


# speedup vs baseline: 2.4874x; 2.4874x over previous
import functools
import jax
import jax.numpy as jnp
import numpy as np
from jax import lax
from jax.experimental import pallas as pl
from jax.experimental.pallas import tpu as pltpu, tpu_sc as plsc

_SCALE = float(2.0 ** -2.5)

_B, _T = 128, 32768
_F = _T // 32
_RPW = 4
_mesh = plsc.VectorSubcoreMesh(core_axis_name="c", subcore_axis_name="s")

_REL = np.concatenate(
    [
        np.array([33 * l + j - 32 * ((l + j) >= 32) for l in range(16)], np.int32)
        for j in range(32)
    ]
)


@functools.partial(
    pl.kernel,
    out_type=jax.ShapeDtypeStruct((_B, _F), jnp.float32),
    mesh=_mesh,
    scratch_types=[
        pltpu.VMEM((_T,), jnp.float32),
        pltpu.VMEM((_T,), jnp.float32),
        pltpu.VMEM((_F,), jnp.float32),
        pltpu.VMEM((32 * 16,), jnp.int32),
        pltpu.SemaphoreType.DMA,
        pltpu.SemaphoreType.DMA,
    ],
    compiler_params=pltpu.CompilerParams(needs_layout_passes=False),
)
def _sc_reduce(x_hbm, rel_hbm, o_hbm, buf_a, buf_b, outb, relv, sem_a, sem_b):
    wid = lax.axis_index("s") * 2 + lax.axis_index("c")
    r0 = wid * _RPW
    pltpu.sync_copy(rel_hbm, relv)
    rel = [relv[pl.ds(j * 16, 16)] for j in range(32)]
    bufs = [buf_a, buf_b]
    sems = [sem_a, sem_b]
    copies = [None, None]
    copies[0] = pltpu.async_copy(x_hbm.at[r0], buf_a, sem_a)
    for r in range(_RPW):
        buf = bufs[r % 2]
        copies[r % 2].wait()
        if r + 1 < _RPW:
            copies[(r + 1) % 2] = pltpu.async_copy(
                x_hbm.at[r0 + r + 1], bufs[(r + 1) % 2], sems[(r + 1) % 2]
            )

        def g_body(g, _):
            base = g * 512
            a0 = plsc.load_gather(buf, [base + rel[0]])
            a1 = plsc.load_gather(buf, [base + rel[1]])
            a2 = plsc.load_gather(buf, [base + rel[2]])
            a3 = plsc.load_gather(buf, [base + rel[3]])
            for j in range(4, 32, 4):
                a0 = a0 + plsc.load_gather(buf, [base + rel[j]])
                a1 = a1 + plsc.load_gather(buf, [base + rel[j + 1]])
                a2 = a2 + plsc.load_gather(buf, [base + rel[j + 2]])
                a3 = a3 + plsc.load_gather(buf, [base + rel[j + 3]])
            outb[pl.ds(g * 16, 16)] = ((a0 + a1) + (a2 + a3)) * _SCALE
            return 0

        lax.fori_loop(0, _F // 16, g_body, 0)
        pltpu.sync_copy(outb, o_hbm.at[r0 + r])


def kernel(x):
    return _sc_reduce(x, jnp.asarray(_REL))[:, :, None]


# speedup vs baseline: 2.9855x; 1.2003x over previous
import functools
import jax
import jax.numpy as jnp
import numpy as np
from jax import lax
from jax.experimental import pallas as pl
from jax.experimental.pallas import tpu as pltpu, tpu_sc as plsc

_SCALE = float(2.0 ** -2.5)

_B, _T = 128, 32768
_F = _T // 32
_TC_ROWS = 96
_SC_ROWS = _B - _TC_ROWS
_mesh = plsc.VectorSubcoreMesh(core_axis_name="c", subcore_axis_name="s")

_REL = np.concatenate(
    [
        np.array([33 * l + j - 32 * ((l + j) >= 32) for l in range(16)], np.int32)
        for j in range(32)
    ]
)


def _tc_body(x_ref, o_ref):
    xb = x_ref[...]
    xt = jnp.transpose(xb)
    s = xt.reshape(xt.shape[0] // 32, 32, xt.shape[1]).sum(axis=1) * _SCALE
    o_ref[...] = jnp.transpose(s)


def _tc_part(x):
    CB = 16384
    return pl.pallas_call(
        _tc_body,
        grid=(_T // CB,),
        in_specs=[pl.BlockSpec((_TC_ROWS, CB), lambda i: (0, i))],
        out_specs=pl.BlockSpec((_TC_ROWS, CB // 32), lambda i: (0, i)),
        out_shape=jax.ShapeDtypeStruct((_TC_ROWS, _F), jnp.float32),
        compiler_params=pltpu.CompilerParams(
            dimension_semantics=("parallel",),
        ),
    )(x)


@functools.partial(
    pl.kernel,
    out_type=jax.ShapeDtypeStruct((_SC_ROWS, _F), jnp.float32),
    mesh=_mesh,
    scratch_types=[
        pltpu.VMEM((_T,), jnp.float32),
        pltpu.VMEM((_F,), jnp.float32),
        pltpu.VMEM((32 * 16,), jnp.int32),
        pltpu.SemaphoreType.DMA,
    ],
    compiler_params=pltpu.CompilerParams(needs_layout_passes=False),
)
def _sc_reduce(x_hbm, rel_hbm, o_hbm, buf, outb, relv, sem):
    wid = lax.axis_index("s") * 2 + lax.axis_index("c")
    row = _TC_ROWS + wid
    pltpu.sync_copy(rel_hbm, relv)
    rel = [relv[pl.ds(j * 16, 16)] for j in range(32)]
    pltpu.async_copy(x_hbm.at[row], buf, sem).wait()

    def g_body(g, _):
        base = g * 512
        a0 = plsc.load_gather(buf, [base + rel[0]])
        a1 = plsc.load_gather(buf, [base + rel[1]])
        a2 = plsc.load_gather(buf, [base + rel[2]])
        a3 = plsc.load_gather(buf, [base + rel[3]])
        for j in range(4, 32, 4):
            a0 = a0 + plsc.load_gather(buf, [base + rel[j]])
            a1 = a1 + plsc.load_gather(buf, [base + rel[j + 1]])
            a2 = a2 + plsc.load_gather(buf, [base + rel[j + 2]])
            a3 = a3 + plsc.load_gather(buf, [base + rel[j + 3]])
        outb[pl.ds(g * 16, 16)] = ((a0 + a1) + (a2 + a3)) * _SCALE
        return 0

    lax.fori_loop(0, _F // 16, g_body, 0)
    pltpu.sync_copy(outb, o_hbm.at[wid])


def kernel(x):
    top = _tc_part(x)
    bot = _sc_reduce(x, jnp.asarray(_REL))
    return jnp.concatenate([top, bot], axis=0)[:, :, None]
